# Initial kernel scaffold; baseline (speedup 1.0000x reference)
import jax
import jax.numpy as jnp
from jax.experimental import pallas as pl


def kernel(x, edge_index, W1, b1, W2, b2):
    raise NotImplementedError("write your pallas kernel here")


# speedup vs baseline: 12.1751x; 12.1751x over previous
import functools

import jax
import jax.numpy as jnp
from jax import lax
from jax.experimental import pallas as pl
from jax.experimental.pallas import tpu as pltpu
from jax.experimental.pallas import tpu_sc as plsc

NC = 2
NS = 16
NW = NC * NS
CHUNK = 128
DEGW = 128

_mesh = plsc.VectorSubcoreMesh(core_axis_name="c", subcore_axis_name="s")


def _make_deg_kernel(npad, ch):
    rows_per_tile = npad // NS

    @functools.partial(
        pl.kernel,
        out_type=jax.ShapeDtypeStruct((NC, npad, DEGW), jnp.float32),
        mesh=_mesh,
        scratch_types=[
            pltpu.VMEM((ch, CHUNK), jnp.int32),
            pltpu.VMEM((CHUNK, DEGW), jnp.float32),
            pltpu.VMEM_SHARED((npad, DEGW), jnp.float32),
        ],
    )
    def deg_kernel(dst_hbm, ones_hbm, zdeg_hbm, out_hbm, idx_v, ones_v, acc_sh):
        c = lax.axis_index("c")
        s = lax.axis_index("s")
        pltpu.sync_copy(ones_hbm, ones_v)
        pltpu.sync_copy(zdeg_hbm, acc_sh.at[pl.ds(s * rows_per_tile, rows_per_tile)])
        plsc.subcore_barrier()

        pltpu.sync_copy(dst_hbm.at[c, s], idx_v)

        def body(j, _):
            pltpu.sync_copy(ones_v, acc_sh.at[idx_v.at[j]], add=True)
            return 0

        lax.fori_loop(0, ch, body, 0)
        plsc.subcore_barrier()
        pltpu.sync_copy(
            acc_sh.at[pl.ds(s * rows_per_tile, rows_per_tile)],
            out_hbm.at[c, pl.ds(s * rows_per_tile, rows_per_tile)],
        )

    return deg_kernel


def _make_scatter_kernel(npad, d, ch):
    rows_per_tile = npad // NS

    @functools.partial(
        pl.kernel,
        out_type=jax.ShapeDtypeStruct((NC, npad, d), jnp.float32),
        mesh=_mesh,
        scratch_types=[
            pltpu.VMEM((ch, CHUNK), jnp.int32),
            pltpu.VMEM((ch, CHUNK), jnp.int32),
            pltpu.VMEM((CHUNK, d), jnp.float32),
            pltpu.VMEM_SHARED((npad, d), jnp.float32),
            pltpu.SemaphoreType.DMA,
        ],
    )
    def scat_kernel(hs_hbm, src_hbm, dst_hbm, zrows_hbm, out_hbm, src_v, dst_v,
                    rows_v, acc_sh, sem):
        c = lax.axis_index("c")
        s = lax.axis_index("s")
        pltpu.sync_copy(zrows_hbm, acc_sh.at[pl.ds(s * rows_per_tile, rows_per_tile)])
        plsc.subcore_barrier()

        pltpu.sync_copy(src_hbm.at[c, s], src_v)
        pltpu.sync_copy(dst_hbm.at[c, s], dst_v)

        def body(j, _):
            pltpu.async_copy(hs_hbm.at[src_v.at[j]], rows_v, sem).wait()
            pltpu.sync_copy(rows_v, acc_sh.at[dst_v.at[j]], add=True)
            return 0

        lax.fori_loop(0, ch, body, 0)
        plsc.subcore_barrier()
        pltpu.sync_copy(
            acc_sh.at[pl.ds(s * rows_per_tile, rows_per_tile)],
            out_hbm.at[c, pl.ds(s * rows_per_tile, rows_per_tile)],
        )

    return scat_kernel


def _dinv(d0_ref, d1_ref):
    deg = 1.0 + d0_ref[:, 0:1] + d1_ref[:, 0:1]
    return lax.rsqrt(deg)


def _tc1_body(x_ref, w_ref, d0_ref, d1_ref, hs_ref):
    h = jnp.dot(x_ref[:], w_ref[:], preferred_element_type=jnp.float32)
    hs_ref[:] = h * _dinv(d0_ref, d1_ref)


def _tc2_body(a0_ref, a1_ref, hs_ref, d0_ref, d1_ref, b_ref, w_ref, out_ref):
    dinv = _dinv(d0_ref, d1_ref)
    h1 = dinv * (a0_ref[:] + a1_ref[:] + hs_ref[:]) + b_ref[:]
    h1 = jnp.maximum(h1, 0.0)
    out_ref[:] = jnp.dot(h1, w_ref[:], preferred_element_type=jnp.float32) * dinv


def _tc3_body(a0_ref, a1_ref, hs_ref, d0_ref, d1_ref, b_ref, out_ref):
    dinv = _dinv(d0_ref, d1_ref)
    out_ref[:] = dinv * (a0_ref[:] + a1_ref[:] + hs_ref[:]) + b_ref[:]


def _row_spec(blk, d):
    return pl.BlockSpec((blk, d), lambda i: (i, 0))


def _full_spec(shape):
    return pl.BlockSpec(shape, lambda i: tuple(0 for _ in shape))


def kernel(x, edge_index, W1, b1, W2, b2):
    n, d_in = x.shape
    d_hid = W1.shape[1]
    d_out = W2.shape[1]
    e = edge_index.shape[1]

    blk = 1024
    npad = ((n + blk - 1) // blk) * blk
    ch = (e + NW * CHUNK - 1) // (NW * CHUNK)
    epad = NW * ch * CHUNK

    src = edge_index[0].astype(jnp.int32)
    dst = edge_index[1].astype(jnp.int32)
    pad_idx = jnp.full((epad - e,), n, dtype=jnp.int32)
    src_p = jnp.concatenate([src, pad_idx]).reshape(NC, NS, ch, CHUNK)
    dst_p = jnp.concatenate([dst, pad_idx]).reshape(NC, NS, ch, CHUNK)
    x_p = jnp.concatenate([x, jnp.zeros((npad - n, d_in), x.dtype)], axis=0)

    rows_per_tile = npad // NS
    ones_rows = jnp.ones((CHUNK, DEGW), jnp.float32)
    zrows = jnp.zeros((rows_per_tile, d_hid), jnp.float32)

    deg2 = _make_deg_kernel(npad, ch)(dst_p, ones_rows, zrows)
    d0, d1 = deg2[0], deg2[1]

    grid = npad // blk
    hs1 = pl.pallas_call(
        _tc1_body,
        grid=(grid,),
        in_specs=[
            _row_spec(blk, d_in),
            _full_spec((d_in, d_hid)),
            _row_spec(blk, DEGW),
            _row_spec(blk, DEGW),
        ],
        out_specs=_row_spec(blk, d_hid),
        out_shape=jax.ShapeDtypeStruct((npad, d_hid), jnp.float32),
    )(x_p, W1, d0, d1)

    scat = _make_scatter_kernel(npad, d_hid, ch)
    acc1 = scat(hs1, src_p, dst_p, zrows)

    hs2 = pl.pallas_call(
        _tc2_body,
        grid=(grid,),
        in_specs=[
            _row_spec(blk, d_hid),
            _row_spec(blk, d_hid),
            _row_spec(blk, d_hid),
            _row_spec(blk, DEGW),
            _row_spec(blk, DEGW),
            _full_spec((1, d_hid)),
            _full_spec((d_hid, d_out)),
        ],
        out_specs=_row_spec(blk, d_out),
        out_shape=jax.ShapeDtypeStruct((npad, d_out), jnp.float32),
    )(acc1[0], acc1[1], hs1, d0, d1, b1.reshape(1, d_hid), W2)

    acc2 = scat(hs2, src_p, dst_p, zrows)

    out = pl.pallas_call(
        _tc3_body,
        grid=(grid,),
        in_specs=[
            _row_spec(blk, d_out),
            _row_spec(blk, d_out),
            _row_spec(blk, d_out),
            _row_spec(blk, DEGW),
            _row_spec(blk, DEGW),
            _full_spec((1, d_out)),
        ],
        out_specs=_row_spec(blk, d_out),
        out_shape=jax.ShapeDtypeStruct((npad, d_out), jnp.float32),
    )(acc2[0], acc2[1], hs2, d0, d1, b2.reshape(1, d_out))

    return out[:n]
